# Initial kernel scaffold; baseline (speedup 1.0000x reference)
import jax
import jax.numpy as jnp
from jax.experimental import pallas as pl


def kernel(feat, proposals, image_shape, target, fc1_w, fc1_b, fc2_w, fc2_b, cls_w, cls_b, reg_w, reg_b):
    raise NotImplementedError("write your pallas kernel here")


# speedup vs baseline: 1.0272x; 1.0272x over previous
import math

import jax
import jax.numpy as jnp
from jax.experimental import pallas as pl
from jax.experimental.pallas import tpu as pltpu

_C = 256
_HF = 50
_WF = 50
_N = 2000
_P = 7
_D = 1024
_NCLS = 81
_IMG = 800
_SCORE_THR = 0.01
_MIN_SIZE = 16.0
_NMS_THR = 0.5
_TOPK = 100
_PRE_NMS_K = 500

_NPAD = 2048
_MBLK = 256
_KBLK = 1792


def _fc1_body(x_ref, w_ref, b_ref, o_ref):
    k = pl.program_id(1)
    nk = pl.num_programs(1)

    @pl.when(k == 0)
    def _():
        o_ref[...] = jnp.zeros_like(o_ref)

    o_ref[...] += jax.lax.dot_general(
        x_ref[...], w_ref[...], (((1,), (1,)), ((), ())),
        preferred_element_type=jnp.float32)

    @pl.when(k == nk - 1)
    def _():
        o_ref[...] = jnp.maximum(o_ref[...] + b_ref[...], 0.0)


def _head_body(x1_ref, w2_ref, b2_ref, clsw_ref, clsb_ref,
               dxw_ref, dyw_ref, dww_ref, dhw_ref,
               dxb_ref, dyb_ref, dwb_ref, dhb_ref,
               prop_ref, imghw_ref,
               sc_ref, bx1_ref, by1_ref, bx2_ref, by2_ref):
    m = pl.program_id(0)

    def mm(x, w):
        return jax.lax.dot_general(
            x, w, (((1,), (1,)), ((), ())), preferred_element_type=jnp.float32)

    x2 = jnp.maximum(mm(x1_ref[...], w2_ref[...]) + b2_ref[...], 0.0)
    logits = mm(x2, clsw_ref[...]) + clsb_ref[...]
    dx = mm(x2, dxw_ref[...]) + dxb_ref[...]
    dy = mm(x2, dyw_ref[...]) + dyb_ref[...]
    dw = mm(x2, dww_ref[...]) + dwb_ref[...]
    dh = mm(x2, dhw_ref[...]) + dhb_ref[...]

    lmax = jnp.max(logits, axis=-1, keepdims=True)
    unn = jnp.exp(logits - lmax)
    scores = unn / jnp.sum(unn, axis=-1, keepdims=True)
    scores = scores[:, 1:]

    h_img = imghw_ref[0, 0]
    w_img = imghw_ref[0, 1]

    px1 = prop_ref[:, 0:1]
    py1 = prop_ref[:, 1:2]
    px2 = prop_ref[:, 2:3]
    py2 = prop_ref[:, 3:4]
    pw = px2 - px1
    ph_ = py2 - py1
    pcx = px1 + 0.5 * pw
    pcy = py1 + 0.5 * ph_

    dmax = math.log(1000.0 / 16.0)
    dw = jnp.minimum(dw, dmax)
    dh = jnp.minimum(dh, dmax)
    cx = dx * pw + pcx
    cy = dy * ph_ + pcy
    bw = jnp.exp(dw) * pw
    bh = jnp.exp(dh) * ph_
    bx1 = jnp.clip(cx - 0.5 * bw, 0.0, w_img)
    by1 = jnp.clip(cy - 0.5 * bh, 0.0, h_img)
    bx2 = jnp.clip(cx + 0.5 * bw, 0.0, w_img)
    by2 = jnp.clip(cy + 0.5 * bh, 0.0, h_img)

    ws = bx2 - bx1
    hs = by2 - by1
    valid = (scores > _SCORE_THR) & (ws >= _MIN_SIZE) & (hs >= _MIN_SIZE)
    sc = jnp.where(valid, scores, -1.0)

    row = jax.lax.broadcasted_iota(jnp.int32, sc.shape, 0) + m * _MBLK
    sc = jnp.where(row < _N, sc, -2.0)

    sc_ref[...] = sc
    bx1_ref[...] = bx1
    by1_ref[...] = by1
    bx2_ref[...] = bx2
    by2_ref[...] = by2


def _iou_1va(box, boxes):
    ix1 = jnp.maximum(box[0], boxes[:, 0])
    iy1 = jnp.maximum(box[1], boxes[:, 1])
    ix2 = jnp.minimum(box[2], boxes[:, 2])
    iy2 = jnp.minimum(box[3], boxes[:, 3])
    inter = jnp.maximum(ix2 - ix1, 0.0) * jnp.maximum(iy2 - iy1, 0.0)
    a1 = jnp.maximum(box[2] - box[0], 0.0) * jnp.maximum(box[3] - box[1], 0.0)
    a2 = (jnp.maximum(boxes[:, 2] - boxes[:, 0], 0.0)
          * jnp.maximum(boxes[:, 3] - boxes[:, 1], 0.0))
    return inter / (a1 + a2 - inter + 1e-9)


def kernel(feat, proposals, image_shape, target, fc1_w, fc1_b, fc2_w, fc2_b,
           cls_w, cls_b, reg_w, reg_b):
    h_img = image_shape[0].astype(jnp.float32)
    w_img = image_shape[1].astype(jnp.float32)
    scale = feat.shape[-1] / w_img
    fmap = feat[0]
    rois = proposals * scale
    rx1, ry1 = rois[:, 0], rois[:, 1]
    rw = jnp.maximum(rois[:, 2] - rx1, 1.0)
    rh = jnp.maximum(rois[:, 3] - ry1, 1.0)
    px = rx1[:, None] + (jnp.arange(_P, dtype=jnp.float32) + 0.5) * (rw / _P)[:, None]
    py = ry1[:, None] + (jnp.arange(_P, dtype=jnp.float32) + 0.5) * (rh / _P)[:, None]
    xi = jnp.clip(jnp.floor(px), 0, _WF - 1).astype(jnp.int32)
    yi = jnp.clip(jnp.floor(py), 0, _HF - 1).astype(jnp.int32)

    pooled = fmap[:, yi[:, :, None], xi[:, None, :]]
    x = jnp.transpose(pooled, (1, 0, 2, 3)).reshape(_N, -1)
    x = jnp.pad(x, ((0, _NPAD - _N), (0, 0)))

    fin = _C * _P * _P
    x1 = pl.pallas_call(
        _fc1_body,
        grid=(_NPAD // _MBLK, fin // _KBLK),
        in_specs=[
            pl.BlockSpec((_MBLK, _KBLK), lambda m, k: (m, k)),
            pl.BlockSpec((_D, _KBLK), lambda m, k: (0, k)),
            pl.BlockSpec((1, _D), lambda m, k: (0, 0)),
        ],
        out_specs=pl.BlockSpec((_MBLK, _D), lambda m, k: (m, 0)),
        out_shape=jax.ShapeDtypeStruct((_NPAD, _D), jnp.float32),
    )(x, fc1_w, fc1_b.reshape(1, _D))

    prop_pad = jnp.pad(proposals, ((0, _NPAD - _N), (0, 0)))
    imghw = jnp.stack([h_img, w_img]).reshape(1, 2)

    reg_w4 = reg_w.reshape(_NCLS, 4, _D)
    reg_b4 = reg_b.reshape(_NCLS, 4)
    dxw, dyw, dww, dhw = (reg_w4[1:, 0], reg_w4[1:, 1],
                          reg_w4[1:, 2], reg_w4[1:, 3])
    dxb, dyb, dwb, dhb = (reg_b4[1:, 0].reshape(1, -1),
                          reg_b4[1:, 1].reshape(1, -1),
                          reg_b4[1:, 2].reshape(1, -1),
                          reg_b4[1:, 3].reshape(1, -1))

    nm = _NPAD // _MBLK
    wspec = lambda shape: pl.BlockSpec(shape, lambda m: tuple(0 for _ in shape))
    outs = pl.pallas_call(
        _head_body,
        grid=(nm,),
        in_specs=[
            pl.BlockSpec((_MBLK, _D), lambda m: (m, 0)),
            wspec((_D, _D)),
            wspec((1, _D)),
            wspec((_NCLS, _D)),
            wspec((1, _NCLS)),
            wspec((_NCLS - 1, _D)),
            wspec((_NCLS - 1, _D)),
            wspec((_NCLS - 1, _D)),
            wspec((_NCLS - 1, _D)),
            wspec((1, _NCLS - 1)),
            wspec((1, _NCLS - 1)),
            wspec((1, _NCLS - 1)),
            wspec((1, _NCLS - 1)),
            pl.BlockSpec((_MBLK, 4), lambda m: (m, 0)),
            wspec((1, 2)),
        ],
        out_specs=[pl.BlockSpec((_MBLK, _NCLS - 1), lambda m: (m, 0))] * 5,
        out_shape=[jax.ShapeDtypeStruct((_NPAD, _NCLS - 1), jnp.float32)] * 5,
    )(x1, fc2_w, fc2_b.reshape(1, _D), cls_w, cls_b.reshape(1, _NCLS),
      dxw, dyw, dww, dhw, dxb, dyb, dwb, dhb, prop_pad, imghw)
    sc, bx1, by1, bx2, by2 = outs

    scores_f = sc[:_N].reshape(-1)
    top_s, top_i = jax.lax.top_k(scores_f, _PRE_NMS_K)

    labels_f = (top_i % (_NCLS - 1) + 1).astype(jnp.float32)
    offset = labels_f * (jnp.maximum(w_img, h_img) + 1.0)
    fx1 = bx1[:_N].reshape(-1)
    fy1 = by1[:_N].reshape(-1)
    fx2 = bx2[:_N].reshape(-1)
    fy2 = by2[:_N].reshape(-1)
    cand = jnp.stack([fx1[top_i] + offset, fy1[top_i] + offset,
                      fx2[top_i] + offset, fy2[top_i] + offset], axis=-1)

    idx = jnp.arange(_PRE_NMS_K)
    sup0 = top_s <= 0.0

    def body(i, sup):
        ious = _iou_1va(cand[i], cand)
        return sup | ((~sup[i]) & (ious > _NMS_THR) & (idx > i))

    sup = jax.lax.fori_loop(0, _PRE_NMS_K, body, sup0)
    kept = jnp.where(sup, -1.0, top_s)
    fin_s, fi = jax.lax.top_k(kept, _TOPK)
    sel = top_i[fi]
    out_boxes = jnp.stack([fx1[sel], fy1[sel], fx2[sel], fy2[sel]], axis=-1)
    out_labels = (sel % (_NCLS - 1) + 1)
    return out_boxes, fin_s, out_labels


# speedup vs baseline: 2.8734x; 2.7973x over previous
import math

import jax
import jax.numpy as jnp
from jax.experimental import pallas as pl
from jax.experimental.pallas import tpu as pltpu

_C = 256
_HF = 50
_WF = 50
_N = 2000
_P = 7
_D = 1024
_NCLS = 81
_IMG = 800
_SCORE_THR = 0.01
_MIN_SIZE = 16.0
_NMS_THR = 0.5
_TOPK = 100
_PRE_NMS_K = 500
_KPAD = 512

_NPAD = 2048
_MBLK = 256
_KBLK = 1792


def _fc1_body(x_ref, w_ref, b_ref, o_ref):
    k = pl.program_id(1)
    nk = pl.num_programs(1)

    @pl.when(k == 0)
    def _():
        o_ref[...] = jnp.zeros_like(o_ref)

    o_ref[...] += jax.lax.dot_general(
        x_ref[...], w_ref[...], (((1,), (1,)), ((), ())),
        preferred_element_type=jnp.float32)

    @pl.when(k == nk - 1)
    def _():
        o_ref[...] = jnp.maximum(o_ref[...] + b_ref[...], 0.0)


def _head_body(x1_ref, w2_ref, b2_ref, clsw_ref, clsb_ref,
               dxw_ref, dyw_ref, dww_ref, dhw_ref,
               dxb_ref, dyb_ref, dwb_ref, dhb_ref,
               prop_ref, imghw_ref,
               sc_ref, bx1_ref, by1_ref, bx2_ref, by2_ref):
    def mm(x, w):
        return jax.lax.dot_general(
            x, w, (((1,), (1,)), ((), ())), preferred_element_type=jnp.float32)

    x2 = jnp.maximum(mm(x1_ref[...], w2_ref[...]) + b2_ref[...], 0.0)
    logits = mm(x2, clsw_ref[...]) + clsb_ref[...]
    dx = mm(x2, dxw_ref[...]) + dxb_ref[...]
    dy = mm(x2, dyw_ref[...]) + dyb_ref[...]
    dw = mm(x2, dww_ref[...]) + dwb_ref[...]
    dh = mm(x2, dhw_ref[...]) + dhb_ref[...]

    lmax = jnp.max(logits, axis=-1, keepdims=True)
    unn = jnp.exp(logits - lmax)
    scores = unn / jnp.sum(unn, axis=-1, keepdims=True)
    scores = scores[:, 1:]

    h_img = imghw_ref[0, 0]
    w_img = imghw_ref[0, 1]

    px1 = prop_ref[:, 0:1]
    py1 = prop_ref[:, 1:2]
    px2 = prop_ref[:, 2:3]
    py2 = prop_ref[:, 3:4]
    pw = px2 - px1
    ph_ = py2 - py1
    pcx = px1 + 0.5 * pw
    pcy = py1 + 0.5 * ph_

    dmax = math.log(1000.0 / 16.0)
    dw = jnp.minimum(dw, dmax)
    dh = jnp.minimum(dh, dmax)
    cx = dx * pw + pcx
    cy = dy * ph_ + pcy
    bw = jnp.exp(dw) * pw
    bh = jnp.exp(dh) * ph_
    bx1 = jnp.clip(cx - 0.5 * bw, 0.0, w_img)
    by1 = jnp.clip(cy - 0.5 * bh, 0.0, h_img)
    bx2 = jnp.clip(cx + 0.5 * bw, 0.0, w_img)
    by2 = jnp.clip(cy + 0.5 * bh, 0.0, h_img)

    ws = bx2 - bx1
    hs = by2 - by1
    valid = (scores > _SCORE_THR) & (ws >= _MIN_SIZE) & (hs >= _MIN_SIZE)
    sc = jnp.where(valid, scores, -1.0)

    row = jax.lax.broadcasted_iota(jnp.int32, sc.shape, 0)
    sc = jnp.where(row < _N, sc, -2.0)

    sc_ref[...] = sc
    bx1_ref[...] = bx1
    by1_ref[...] = by1
    bx2_ref[...] = bx2
    by2_ref[...] = by2


def _nms_body(bx1_ref, by1_ref, bx2_ref, by2_ref,
              cx1_ref, cy1_ref, cx2_ref, cy2_ref,
              ts_ref, kept_ref, sup_ref):
    ox1 = bx1_ref[...]
    oy1 = by1_ref[...]
    ox2 = bx2_ref[...]
    oy2 = by2_ref[...]
    ts = ts_ref[...]
    idx = jax.lax.broadcasted_iota(jnp.int32, ox1.shape, 1)

    sup_ref[...] = jnp.where(ts <= 0.0, 1.0, 0.0)

    def body(i, _):
        px1 = cx1_ref[i, 0]
        py1 = cy1_ref[i, 0]
        px2 = cx2_ref[i, 0]
        py2 = cy2_ref[i, 0]
        sup = sup_ref[...]
        sup_i = jnp.max(jnp.where(idx == i, sup, 0.0))
        ix1 = jnp.maximum(px1, ox1)
        iy1 = jnp.maximum(py1, oy1)
        ix2 = jnp.minimum(px2, ox2)
        iy2 = jnp.minimum(py2, oy2)
        inter = jnp.maximum(ix2 - ix1, 0.0) * jnp.maximum(iy2 - iy1, 0.0)
        a1 = (jnp.maximum(px2 - px1, 0.0) * jnp.maximum(py2 - py1, 0.0))
        a2 = (jnp.maximum(ox2 - ox1, 0.0) * jnp.maximum(oy2 - oy1, 0.0))
        iou = inter / (a1 + a2 - inter + 1e-9)
        cond = (sup_i == 0.0) & (iou > _NMS_THR) & (idx > i)
        sup_ref[...] = jnp.where(cond, 1.0, sup)
        return 0

    jax.lax.fori_loop(0, _PRE_NMS_K, body, 0)
    kept_ref[...] = jnp.where(sup_ref[...] > 0.0, -1.0, ts)


def kernel(feat, proposals, image_shape, target, fc1_w, fc1_b, fc2_w, fc2_b,
           cls_w, cls_b, reg_w, reg_b):
    h_img = image_shape[0].astype(jnp.float32)
    w_img = image_shape[1].astype(jnp.float32)
    scale = feat.shape[-1] / w_img
    fmap = feat[0]
    rois = proposals * scale
    rx1, ry1 = rois[:, 0], rois[:, 1]
    rw = jnp.maximum(rois[:, 2] - rx1, 1.0)
    rh = jnp.maximum(rois[:, 3] - ry1, 1.0)
    px = rx1[:, None] + (jnp.arange(_P, dtype=jnp.float32) + 0.5) * (rw / _P)[:, None]
    py = ry1[:, None] + (jnp.arange(_P, dtype=jnp.float32) + 0.5) * (rh / _P)[:, None]
    xi = jnp.clip(jnp.floor(px), 0, _WF - 1).astype(jnp.int32)
    yi = jnp.clip(jnp.floor(py), 0, _HF - 1).astype(jnp.int32)

    pooled = fmap[:, yi[:, :, None], xi[:, None, :]]
    x = jnp.transpose(pooled, (1, 0, 2, 3)).reshape(_N, -1)
    x = jnp.pad(x, ((0, _NPAD - _N), (0, 0)))

    fin = _C * _P * _P
    x1 = pl.pallas_call(
        _fc1_body,
        grid=(_NPAD // _MBLK, fin // _KBLK),
        in_specs=[
            pl.BlockSpec((_MBLK, _KBLK), lambda m, k: (m, k)),
            pl.BlockSpec((_D, _KBLK), lambda m, k: (0, k)),
            pl.BlockSpec((1, _D), lambda m, k: (0, 0)),
        ],
        out_specs=pl.BlockSpec((_MBLK, _D), lambda m, k: (m, 0)),
        out_shape=jax.ShapeDtypeStruct((_NPAD, _D), jnp.float32),
    )(x, fc1_w, fc1_b.reshape(1, _D))

    prop_pad = jnp.pad(proposals, ((0, _NPAD - _N), (0, 0)))
    imghw = jnp.stack([h_img, w_img]).reshape(1, 2)

    reg_w4 = reg_w.reshape(_NCLS, 4, _D)
    reg_b4 = reg_b.reshape(_NCLS, 4)
    dxw, dyw, dww, dhw = (reg_w4[1:, 0], reg_w4[1:, 1],
                          reg_w4[1:, 2], reg_w4[1:, 3])
    dxb, dyb, dwb, dhb = (reg_b4[1:, 0].reshape(1, -1),
                          reg_b4[1:, 1].reshape(1, -1),
                          reg_b4[1:, 2].reshape(1, -1),
                          reg_b4[1:, 3].reshape(1, -1))

    full = lambda shape: pl.BlockSpec(shape, lambda: tuple(0 for _ in shape))
    outs = pl.pallas_call(
        _head_body,
        grid=(),
        in_specs=[
            full((_NPAD, _D)),
            full((_D, _D)),
            full((1, _D)),
            full((_NCLS, _D)),
            full((1, _NCLS)),
            full((_NCLS - 1, _D)),
            full((_NCLS - 1, _D)),
            full((_NCLS - 1, _D)),
            full((_NCLS - 1, _D)),
            full((1, _NCLS - 1)),
            full((1, _NCLS - 1)),
            full((1, _NCLS - 1)),
            full((1, _NCLS - 1)),
            full((_NPAD, 4)),
            full((1, 2)),
        ],
        out_specs=[full((_NPAD, _NCLS - 1))] * 5,
        out_shape=[jax.ShapeDtypeStruct((_NPAD, _NCLS - 1), jnp.float32)] * 5,
    )(x1, fc2_w, fc2_b.reshape(1, _D), cls_w, cls_b.reshape(1, _NCLS),
      dxw, dyw, dww, dhw, dxb, dyb, dwb, dhb, prop_pad, imghw)
    sc, bx1, by1, bx2, by2 = outs

    scores_f = sc[:_N].reshape(-1)
    top_s, top_i = jax.lax.top_k(scores_f, _PRE_NMS_K)

    labels_f = (top_i % (_NCLS - 1) + 1).astype(jnp.float32)
    offset = labels_f * (jnp.maximum(w_img, h_img) + 1.0)
    fx1 = bx1[:_N].reshape(-1)
    fy1 = by1[:_N].reshape(-1)
    fx2 = bx2[:_N].reshape(-1)
    fy2 = by2[:_N].reshape(-1)
    cand_x1 = fx1[top_i] + offset
    cand_y1 = fy1[top_i] + offset
    cand_x2 = fx2[top_i] + offset
    cand_y2 = fy2[top_i] + offset

    padv = jnp.full((_KPAD - _PRE_NMS_K,), 0.0, jnp.float32)
    row = lambda v: jnp.concatenate([v, padv]).reshape(1, _KPAD)
    col = lambda v: jnp.concatenate([v, padv]).reshape(_KPAD, 1)
    ts_row = jnp.concatenate(
        [top_s, jnp.full((_KPAD - _PRE_NMS_K,), -2.0, jnp.float32)]
    ).reshape(1, _KPAD)

    kept = pl.pallas_call(
        _nms_body,
        grid=(),
        in_specs=[full((1, _KPAD))] * 4 + [full((_KPAD, 1))] * 4
                 + [full((1, _KPAD))],
        out_specs=full((1, _KPAD)),
        out_shape=jax.ShapeDtypeStruct((1, _KPAD), jnp.float32),
        scratch_shapes=[pltpu.VMEM((1, _KPAD), jnp.float32)],
    )(row(cand_x1), row(cand_y1), row(cand_x2), row(cand_y2),
      col(cand_x1), col(cand_y1), col(cand_x2), col(cand_y2), ts_row)

    kept = kept[0, :_PRE_NMS_K]
    fin_s, fi = jax.lax.top_k(kept, _TOPK)
    sel = top_i[fi]
    out_boxes = jnp.stack([fx1[sel], fy1[sel], fx2[sel], fy2[sel]], axis=-1)
    out_labels = (sel % (_NCLS - 1) + 1)
    return out_boxes, fin_s, out_labels


# speedup vs baseline: 6.0638x; 2.1103x over previous
import functools
import math

import jax
import jax.numpy as jnp
from jax import lax
from jax.experimental import pallas as pl
from jax.experimental.pallas import tpu as pltpu
from jax.experimental.pallas import tpu_sc as plsc

_C = 256
_HF = 50
_WF = 50
_N = 2000
_P = 7
_D = 1024
_NCLS = 81
_IMG = 800
_SCORE_THR = 0.01
_MIN_SIZE = 16.0
_NMS_THR = 0.5
_TOPK = 100
_PRE_NMS_K = 500
_KPAD = 512

_NPAD = 2048
_MBLK = 256
_KBLK = 1792


def _fc1_body(x_ref, w_ref, b_ref, o_ref):
    k = pl.program_id(1)
    nk = pl.num_programs(1)

    @pl.when(k == 0)
    def _():
        o_ref[...] = jnp.zeros_like(o_ref)

    o_ref[...] += jax.lax.dot_general(
        x_ref[...], w_ref[...], (((1,), (1,)), ((), ())),
        preferred_element_type=jnp.float32)

    @pl.when(k == nk - 1)
    def _():
        o_ref[...] = jnp.maximum(o_ref[...] + b_ref[...], 0.0)


def _head_body(x1_ref, w2_ref, b2_ref, clsw_ref, clsb_ref,
               dxw_ref, dyw_ref, dww_ref, dhw_ref,
               dxb_ref, dyb_ref, dwb_ref, dhb_ref,
               prop_ref, imghw_ref,
               sc_ref, bx1_ref, by1_ref, bx2_ref, by2_ref):
    def mm(x, w):
        return jax.lax.dot_general(
            x, w, (((1,), (1,)), ((), ())), preferred_element_type=jnp.float32)

    x2 = jnp.maximum(mm(x1_ref[...], w2_ref[...]) + b2_ref[...], 0.0)
    logits = mm(x2, clsw_ref[...]) + clsb_ref[...]
    dx = mm(x2, dxw_ref[...]) + dxb_ref[...]
    dy = mm(x2, dyw_ref[...]) + dyb_ref[...]
    dw = mm(x2, dww_ref[...]) + dwb_ref[...]
    dh = mm(x2, dhw_ref[...]) + dhb_ref[...]

    lmax = jnp.max(logits, axis=-1, keepdims=True)
    unn = jnp.exp(logits - lmax)
    scores = unn / jnp.sum(unn, axis=-1, keepdims=True)
    scores = scores[:, 1:]

    h_img = imghw_ref[0, 0]
    w_img = imghw_ref[0, 1]

    px1 = prop_ref[:, 0:1]
    py1 = prop_ref[:, 1:2]
    px2 = prop_ref[:, 2:3]
    py2 = prop_ref[:, 3:4]
    pw = px2 - px1
    ph_ = py2 - py1
    pcx = px1 + 0.5 * pw
    pcy = py1 + 0.5 * ph_

    dmax = math.log(1000.0 / 16.0)
    dw = jnp.minimum(dw, dmax)
    dh = jnp.minimum(dh, dmax)
    cx = dx * pw + pcx
    cy = dy * ph_ + pcy
    bw = jnp.exp(dw) * pw
    bh = jnp.exp(dh) * ph_
    bx1 = jnp.clip(cx - 0.5 * bw, 0.0, w_img)
    by1 = jnp.clip(cy - 0.5 * bh, 0.0, h_img)
    bx2 = jnp.clip(cx + 0.5 * bw, 0.0, w_img)
    by2 = jnp.clip(cy + 0.5 * bh, 0.0, h_img)

    ws = bx2 - bx1
    hs = by2 - by1
    valid = (scores > _SCORE_THR) & (ws >= _MIN_SIZE) & (hs >= _MIN_SIZE)
    sc = jnp.where(valid, scores, -1.0)

    row = jax.lax.broadcasted_iota(jnp.int32, sc.shape, 0)
    sc = jnp.where(row < _N, sc, -2.0)

    sc_ref[...] = sc
    bx1_ref[...] = bx1
    by1_ref[...] = by1
    bx2_ref[...] = bx2
    by2_ref[...] = by2


_NW = 32
_GB = _NPAD * _P * _P
_BPW = _GB // _NW
_GCH = 8
_CHROWS = _BPW // _GCH


def _sc_gather(table, idx):
    mesh = plsc.VectorSubcoreMesh(core_axis_name="c", subcore_axis_name="s")

    @functools.partial(
        pl.kernel, mesh=mesh,
        out_type=jax.ShapeDtypeStruct((_GB, _C), jnp.float32),
        scratch_types=[
            pltpu.VMEM((_BPW,), jnp.int32),
            pltpu.VMEM((_CHROWS, _C), jnp.float32),
            pltpu.SemaphoreType.DMA,
        ],
    )
    def k(table_hbm, idx_hbm, out_hbm, idx_v, rows_v, sem):
        wid = lax.axis_index("s") * 2 + lax.axis_index("c")
        base = wid * _BPW
        pltpu.sync_copy(idx_hbm.at[pl.ds(base, _BPW)], idx_v)
        for ch in range(_GCH):
            pltpu.async_copy(
                table_hbm.at[idx_v.at[pl.ds(ch * _CHROWS, _CHROWS)]],
                rows_v, sem).wait()
            pltpu.sync_copy(rows_v,
                            out_hbm.at[pl.ds(base + ch * _CHROWS, _CHROWS)])

    return k(table, idx)


def _nms_body(bx1_ref, by1_ref, bx2_ref, by2_ref,
              cx1_ref, cy1_ref, cx2_ref, cy2_ref,
              ts_ref, kept_ref, sup_ref):
    ox1 = bx1_ref[...]
    oy1 = by1_ref[...]
    ox2 = bx2_ref[...]
    oy2 = by2_ref[...]
    ts = ts_ref[...]
    idx = jax.lax.broadcasted_iota(jnp.int32, ox1.shape, 1)

    sup_ref[...] = jnp.where(ts <= 0.0, 1.0, 0.0)

    def body(i, _):
        px1 = cx1_ref[i, 0]
        py1 = cy1_ref[i, 0]
        px2 = cx2_ref[i, 0]
        py2 = cy2_ref[i, 0]
        sup = sup_ref[...]
        sup_i = jnp.max(jnp.where(idx == i, sup, 0.0))
        ix1 = jnp.maximum(px1, ox1)
        iy1 = jnp.maximum(py1, oy1)
        ix2 = jnp.minimum(px2, ox2)
        iy2 = jnp.minimum(py2, oy2)
        inter = jnp.maximum(ix2 - ix1, 0.0) * jnp.maximum(iy2 - iy1, 0.0)
        a1 = (jnp.maximum(px2 - px1, 0.0) * jnp.maximum(py2 - py1, 0.0))
        a2 = (jnp.maximum(ox2 - ox1, 0.0) * jnp.maximum(oy2 - oy1, 0.0))
        iou = inter / (a1 + a2 - inter + 1e-9)
        cond = (sup_i == 0.0) & (iou > _NMS_THR) & (idx > i)
        sup_ref[...] = jnp.where(cond, 1.0, sup)
        return 0

    jax.lax.fori_loop(0, _PRE_NMS_K, body, 0)
    kept_ref[...] = jnp.where(sup_ref[...] > 0.0, -1.0, ts)


def kernel(feat, proposals, image_shape, target, fc1_w, fc1_b, fc2_w, fc2_b,
           cls_w, cls_b, reg_w, reg_b):
    h_img = image_shape[0].astype(jnp.float32)
    w_img = image_shape[1].astype(jnp.float32)
    scale = feat.shape[-1] / w_img
    fmap = feat[0]
    rois = proposals * scale
    rx1, ry1 = rois[:, 0], rois[:, 1]
    rw = jnp.maximum(rois[:, 2] - rx1, 1.0)
    rh = jnp.maximum(rois[:, 3] - ry1, 1.0)
    px = rx1[:, None] + (jnp.arange(_P, dtype=jnp.float32) + 0.5) * (rw / _P)[:, None]
    py = ry1[:, None] + (jnp.arange(_P, dtype=jnp.float32) + 0.5) * (rh / _P)[:, None]
    xi = jnp.clip(jnp.floor(px), 0, _WF - 1).astype(jnp.int32)
    yi = jnp.clip(jnp.floor(py), 0, _HF - 1).astype(jnp.int32)

    table = fmap.reshape(_C, _HF * _WF).T
    s_idx = (yi[:, :, None] * _WF + xi[:, None, :]).reshape(_N, _P * _P)
    s_idx = jnp.pad(s_idx, ((0, _NPAD - _N), (0, 0))).reshape(-1)
    x = _sc_gather(table, s_idx.astype(jnp.int32))
    x = x.reshape(_NPAD, _P * _P * _C)

    w1p = fc1_w.reshape(_D, _C, _P * _P).transpose(0, 2, 1).reshape(_D, -1)

    fin = _C * _P * _P
    x1 = pl.pallas_call(
        _fc1_body,
        grid=(_NPAD // _MBLK, fin // _KBLK),
        in_specs=[
            pl.BlockSpec((_MBLK, _KBLK), lambda m, k: (m, k)),
            pl.BlockSpec((_D, _KBLK), lambda m, k: (0, k)),
            pl.BlockSpec((1, _D), lambda m, k: (0, 0)),
        ],
        out_specs=pl.BlockSpec((_MBLK, _D), lambda m, k: (m, 0)),
        out_shape=jax.ShapeDtypeStruct((_NPAD, _D), jnp.float32),
    )(x, w1p, fc1_b.reshape(1, _D))

    prop_pad = jnp.pad(proposals, ((0, _NPAD - _N), (0, 0)))
    imghw = jnp.stack([h_img, w_img]).reshape(1, 2)

    reg_w4 = reg_w.reshape(_NCLS, 4, _D)
    reg_b4 = reg_b.reshape(_NCLS, 4)
    dxw, dyw, dww, dhw = (reg_w4[1:, 0], reg_w4[1:, 1],
                          reg_w4[1:, 2], reg_w4[1:, 3])
    dxb, dyb, dwb, dhb = (reg_b4[1:, 0].reshape(1, -1),
                          reg_b4[1:, 1].reshape(1, -1),
                          reg_b4[1:, 2].reshape(1, -1),
                          reg_b4[1:, 3].reshape(1, -1))

    full = lambda shape: pl.BlockSpec(shape, lambda: tuple(0 for _ in shape))
    outs = pl.pallas_call(
        _head_body,
        grid=(),
        in_specs=[
            full((_NPAD, _D)),
            full((_D, _D)),
            full((1, _D)),
            full((_NCLS, _D)),
            full((1, _NCLS)),
            full((_NCLS - 1, _D)),
            full((_NCLS - 1, _D)),
            full((_NCLS - 1, _D)),
            full((_NCLS - 1, _D)),
            full((1, _NCLS - 1)),
            full((1, _NCLS - 1)),
            full((1, _NCLS - 1)),
            full((1, _NCLS - 1)),
            full((_NPAD, 4)),
            full((1, 2)),
        ],
        out_specs=[full((_NPAD, _NCLS - 1))] * 5,
        out_shape=[jax.ShapeDtypeStruct((_NPAD, _NCLS - 1), jnp.float32)] * 5,
    )(x1, fc2_w, fc2_b.reshape(1, _D), cls_w, cls_b.reshape(1, _NCLS),
      dxw, dyw, dww, dhw, dxb, dyb, dwb, dhb, prop_pad, imghw)
    sc, bx1, by1, bx2, by2 = outs

    scores_f = sc[:_N].reshape(-1)
    top_s, top_i = jax.lax.top_k(scores_f, _PRE_NMS_K)

    labels_f = (top_i % (_NCLS - 1) + 1).astype(jnp.float32)
    offset = labels_f * (jnp.maximum(w_img, h_img) + 1.0)
    fx1 = bx1[:_N].reshape(-1)
    fy1 = by1[:_N].reshape(-1)
    fx2 = bx2[:_N].reshape(-1)
    fy2 = by2[:_N].reshape(-1)
    cand_x1 = fx1[top_i] + offset
    cand_y1 = fy1[top_i] + offset
    cand_x2 = fx2[top_i] + offset
    cand_y2 = fy2[top_i] + offset

    padv = jnp.full((_KPAD - _PRE_NMS_K,), 0.0, jnp.float32)
    row = lambda v: jnp.concatenate([v, padv]).reshape(1, _KPAD)
    col = lambda v: jnp.concatenate([v, padv]).reshape(_KPAD, 1)
    ts_row = jnp.concatenate(
        [top_s, jnp.full((_KPAD - _PRE_NMS_K,), -2.0, jnp.float32)]
    ).reshape(1, _KPAD)

    kept = pl.pallas_call(
        _nms_body,
        grid=(),
        in_specs=[full((1, _KPAD))] * 4 + [full((_KPAD, 1))] * 4
                 + [full((1, _KPAD))],
        out_specs=full((1, _KPAD)),
        out_shape=jax.ShapeDtypeStruct((1, _KPAD), jnp.float32),
        scratch_shapes=[pltpu.VMEM((1, _KPAD), jnp.float32)],
    )(row(cand_x1), row(cand_y1), row(cand_x2), row(cand_y2),
      col(cand_x1), col(cand_y1), col(cand_x2), col(cand_y2), ts_row)

    kept = kept[0, :_PRE_NMS_K]
    fin_s, fi = jax.lax.top_k(kept, _TOPK)
    sel = top_i[fi]
    out_boxes = jnp.stack([fx1[sel], fy1[sel], fx2[sel], fy2[sel]], axis=-1)
    out_labels = (sel % (_NCLS - 1) + 1)
    return out_boxes, fin_s, out_labels
